# Initial kernel scaffold; baseline (speedup 1.0000x reference)
import jax
import jax.numpy as jnp
from jax.experimental import pallas as pl


def kernel(x, edge_index, W1, b1, W2, b2):
    raise NotImplementedError("write your pallas kernel here")


# speedup vs baseline: 4.3582x; 4.3582x over previous
import functools

import jax
import jax.numpy as jnp
from jax import lax
from jax.experimental import pallas as pl
from jax.experimental.pallas import tpu as pltpu
from jax.experimental.pallas import tpu_sc as plsc

N_NODES = 10000
N_EDGES = 320000
D = 128

NC = 2
NS = 16
NW = NC * NS
L = 16

NP = 10240
EPT = N_EDGES // NW
CH = 80
NCHUNK = EPT // CH
RPT = NP // NS


def _sc_mesh():
    return plsc.VectorSubcoreMesh(
        core_axis_name="c", subcore_axis_name="s", num_cores=NC, num_subcores=NS
    )


def _zero_fill(buf, nrows):
    def fill_zero(i, carry):
        for j in range(D // L):
            buf[i, pl.ds(j * L, L)] = jnp.zeros((L,), jnp.float32)
        return carry

    lax.fori_loop(0, nrows, fill_zero, 0)


def _zero_acc(rows_v, acc, sid):
    r0 = sid * RPT
    for j in range(RPT // CH):
        pltpu.sync_copy(rows_v, acc.at[pl.ds(r0 + j * CH, CH)])


def _agg_body(h_hbm, src_hbm, dst_hbm, out_hbm, src_v, dst_v, rows_v, acc, sem):
    cid = lax.axis_index("c")
    sid = lax.axis_index("s")
    wid = sid * NC + cid

    _zero_fill(rows_v, CH)
    _zero_acc(rows_v, acc, sid)
    plsc.subcore_barrier()

    def body(k, carry):
        base = wid * EPT + k * CH
        pltpu.sync_copy(src_hbm.at[pl.ds(base, CH)], src_v)
        pltpu.sync_copy(dst_hbm.at[pl.ds(base, CH)], dst_v)
        pltpu.async_copy(h_hbm.at[src_v], rows_v, sem).wait()
        pltpu.sync_copy(rows_v, acc.at[dst_v], add=True)
        return carry

    lax.fori_loop(0, NCHUNK, body, 0)
    plsc.subcore_barrier()

    r0 = sid * RPT
    pltpu.sync_copy(acc.at[pl.ds(r0, RPT)], out_hbm.at[cid, pl.ds(r0, RPT)])


def _agg_call(h, src, dst):
    return pl.kernel(
        _agg_body,
        out_type=jax.ShapeDtypeStruct((NC, NP, D), jnp.float32),
        mesh=_sc_mesh(),
        scratch_types=[
            pltpu.VMEM((CH,), jnp.int32),
            pltpu.VMEM((CH,), jnp.int32),
            pltpu.VMEM((CH, D), jnp.float32),
            pltpu.VMEM_SHARED((NP, D), jnp.float32),
            pltpu.SemaphoreType.DMA,
        ],
    )(h, src, dst)


def _deg_body(src_hbm, dst_hbm, out_hbm, idx_v, ones_v, zb_v, acc):
    cid = lax.axis_index("c")
    sid = lax.axis_index("s")
    wid = sid * NC + cid
    r0 = sid * RPT

    _zero_fill(zb_v, CH)

    def fill_one(i, carry):
        for j in range(D // L):
            ones_v[i, pl.ds(j * L, L)] = jnp.full((L,), 1.0, jnp.float32)
        return carry

    lax.fori_loop(0, CH, fill_one, 0)

    for which, idx_hbm in ((0, src_hbm), (1, dst_hbm)):
        _zero_acc(zb_v, acc, sid)
        plsc.subcore_barrier()

        def body(k, carry):
            base = wid * EPT + k * CH
            pltpu.sync_copy(idx_hbm.at[pl.ds(base, CH)], idx_v)
            pltpu.sync_copy(ones_v, acc.at[idx_v], add=True)
            return carry

        lax.fori_loop(0, NCHUNK, body, 0)
        plsc.subcore_barrier()
        pltpu.sync_copy(
            acc.at[pl.ds(r0, RPT)], out_hbm.at[cid, which, pl.ds(r0, RPT)]
        )
        plsc.subcore_barrier()


def _deg_call(src, dst):
    return pl.kernel(
        _deg_body,
        out_type=jax.ShapeDtypeStruct((NC, 2, NP, D), jnp.float32),
        mesh=_sc_mesh(),
        scratch_types=[
            pltpu.VMEM((CH,), jnp.int32),
            pltpu.VMEM((CH, D), jnp.float32),
            pltpu.VMEM((CH, D), jnp.float32),
            pltpu.VMEM_SHARED((NP, D), jnp.float32),
        ],
    )(src, dst)


def _scale_kernel_body(x_ref, degs_ref, h_ref, ns_ref, nd_ref):
    dout = degs_ref[0, 0, :, 0:1] + degs_ref[1, 0, :, 0:1]
    din = degs_ref[0, 1, :, 0:1] + degs_ref[1, 1, :, 0:1]
    ns = lax.rsqrt(jnp.maximum(dout, 1.0))
    nd = lax.rsqrt(jnp.maximum(din, 1.0))
    h_ref[...] = x_ref[...] * ns
    ns_ref[...] = ns
    nd_ref[...] = nd


def _scale_call(x_pad, degs):
    return pl.pallas_call(
        _scale_kernel_body,
        out_shape=[
            jax.ShapeDtypeStruct((NP, D), jnp.float32),
            jax.ShapeDtypeStruct((NP, 1), jnp.float32),
            jax.ShapeDtypeStruct((NP, 1), jnp.float32),
        ],
    )(x_pad, degs)


def _dense_kernel_body(scale_next, p_ref, nd_ref, ns_ref, w_ref, b_ref, o_ref):
    agg = (p_ref[0] + p_ref[1]) * nd_ref[...]
    y = jnp.dot(agg, w_ref[...], preferred_element_type=jnp.float32)
    y = jnp.maximum(y + b_ref[...], 0.0)
    if scale_next:
        y = y * ns_ref[...]
    o_ref[...] = y


def _dense_call(partials, nd, ns, w, b, scale_next):
    return pl.pallas_call(
        functools.partial(_dense_kernel_body, scale_next),
        out_shape=jax.ShapeDtypeStruct((NP, D), jnp.float32),
    )(partials, nd, ns, w, b)


def kernel(x, edge_index, W1, b1, W2, b2):
    src = edge_index[0]
    dst = edge_index[1]
    x_pad = jnp.pad(x, ((0, NP - N_NODES), (0, 0)))
    b1r = b1.reshape(1, D)
    b2r = b2.reshape(1, D)

    degs = _deg_call(src, dst)
    h1, ns, nd = _scale_call(x_pad, degs)
    p1 = _agg_call(h1, src, dst)
    h2 = _dense_call(p1, nd, ns, W1, b1r, scale_next=True)
    p2 = _agg_call(h2, src, dst)
    out = _dense_call(p2, nd, ns, W2, b2r, scale_next=False)
    return out[:N_NODES]
